# Initial kernel scaffold; baseline (speedup 1.0000x reference)
import jax
import jax.numpy as jnp
from jax.experimental import pallas as pl


def kernel(inputs):
    raise NotImplementedError("write your pallas kernel here")


# speedup vs baseline: 1.6134x; 1.6134x over previous
import functools

import jax
import jax.numpy as jnp
from jax import lax
from jax.experimental import pallas as pl
from jax.experimental.pallas import tpu as pltpu
from jax.experimental.pallas import tpu_sc as plsc

N_EVENTS = 16384
ROWS = 16
COMPS = 3
EV_WORDS = ROWS * COMPS

NC, NS, L = 2, 16, 16
NW = NC * NS
EV_PER_W = N_EVENTS // NW
W_WORDS = EV_PER_W * EV_WORDS
UNROLL = 8


def _sc_body(x_hbm, out_hbm, in_v, out_v):
    wid = lax.axis_index("s") * NC + lax.axis_index("c")
    base_w = wid * W_WORDS
    pltpu.sync_copy(x_hbm.at[pl.ds(base_w, W_WORDS)], in_v)

    i16 = lax.iota(jnp.int32, L)
    idx3 = i16 * 3
    seg = ((i16 >= 6).astype(jnp.int32) + (i16 >= 9).astype(jnp.int32)
           + (i16 >= 12).astype(jnp.int32) + (i16 >= 15).astype(jnp.int32))
    shv = seg << 2
    segstart = jnp.where(i16 < 6, 0,
                jnp.where(i16 < 9, 6,
                 jnp.where(i16 < 12, 9,
                  jnp.where(i16 < 15, 12, 15))))
    segend = jnp.where(i16 < 6, 6,
              jnp.where(i16 < 9, 9,
               jnp.where(i16 < 12, 12,
                jnp.where(i16 < 15, 15, 16))))
    rejc = segend + segstart - 1 - i16
    lane15 = i16 == 15

    def one_event(e):
        base = e * EV_WORDS
        a = idx3 + base
        v0 = plsc.load_gather(in_v, [a])
        v1 = plsc.load_gather(in_v, [a + 1])
        v2 = plsc.load_gather(in_v, [a + 2])
        m = (v0 > 0.0) | lane15
        w = m.astype(jnp.int32) << shv
        cs = plsc.cumsum(w)
        excl = cs - w
        rank = lax.shift_right_logical(excl, shv) & 15
        dst = rank + jnp.where(m, segstart, rejc)
        d0 = dst * 3 + base
        zero = jnp.zeros((L,), jnp.float32)
        plsc.store_scatter(out_v, [d0], jnp.where(m, v0, zero))
        plsc.store_scatter(out_v, [d0 + 1], jnp.where(m, v1, zero))
        plsc.store_scatter(out_v, [d0 + 2], jnp.where(m, v2, zero))

    def body(it, carry):
        for u in range(UNROLL):
            one_event(it * UNROLL + u)
        return carry

    lax.fori_loop(0, EV_PER_W // UNROLL, body, jnp.int32(0))
    pltpu.sync_copy(out_v, out_hbm.at[pl.ds(base_w, W_WORDS)])


@jax.jit
def _reorder(xf):
    mesh = plsc.VectorSubcoreMesh(core_axis_name="c", subcore_axis_name="s",
                                  num_cores=NC, num_subcores=NS)
    return pl.kernel(
        _sc_body,
        out_type=jax.ShapeDtypeStruct((N_EVENTS * EV_WORDS,), jnp.float32),
        mesh=mesh,
        scratch_types=[
            pltpu.VMEM((W_WORDS,), jnp.float32),
            pltpu.VMEM((W_WORDS,), jnp.float32),
        ],
        compiler_params=pltpu.CompilerParams(needs_layout_passes=False),
    )(xf)


def kernel(inputs):
    xf = inputs.reshape(N_EVENTS * EV_WORDS)
    return _reorder(xf).reshape(N_EVENTS, ROWS, COMPS)


# speedup vs baseline: 25.9857x; 16.1065x over previous
import functools

import jax
import jax.numpy as jnp
from jax import lax
from jax.experimental import pallas as pl
from jax.experimental.pallas import tpu as pltpu
from jax.experimental.pallas import tpu_sc as plsc

N_EVENTS = 16384
TOTAL = N_EVENTS * 48

NC, NS, L = 2, 16, 16
NW = NC * NS
EV_PER_W = N_EVENTS // NW
W_WORDS = EV_PER_W * 48
CHUNK = 4096
PLANE = N_EVENTS * 8

SEGS = [(0, 6), (6, 9), (9, 12), (12, 15)]


def _sc_body(x_hbm, out_hbm, in_v, out_v, sem):
    wid = lax.axis_index("s") * NC + lax.axis_index("c")
    wbase = wid * CHUNK

    copies = []
    for p in range(6):
        copies.append(pltpu.async_copy(
            x_hbm.at[pl.ds(p * PLANE + wbase, CHUNK)],
            in_v.at[pl.ds(p * CHUNK, CHUNK)], sem))
    for cp in copies:
        cp.wait()

    i16 = lax.iota(jnp.int32, L)
    one = jnp.ones((L,), jnp.int32)
    zf = jnp.zeros((L,), jnp.float32)
    zi = jnp.zeros((L,), jnp.int32)

    def block(m):
        mbase = (m >> 3) * 1024 + (m & 7) * 16

        def off(i, c):
            return mbase + (c * 2 + (i >> 3)) * CHUNK + (i & 7) * 128

        v = [[in_v[pl.ds(off(i, c), L)] for c in range(3)] for i in range(16)]
        msk = [v[i][0] > 0.0 for i in range(15)]
        mi = [jnp.where(msk[i], one, zi) for i in range(15)]

        rank = [None] * 15
        for s, e in SEGS:
            acc = zi
            for i in range(s, e):
                rank[i] = acc
                acc = acc + mi[i]

        for s, e in SEGS:
            for i in range(s, e):
                dst = rank[i] + jnp.where(msk[i], s, e + s - 1 - i)
                g = ((dst >> 3) << 12) + ((dst & 7) << 7) + (mbase + i16)
                for c in range(3):
                    val = jnp.where(msk[i], v[i][c], zf)
                    plsc.store_scatter(out_v, [g + c * (2 * CHUNK)], val)
        for c in range(3):
            out_v[pl.ds(off(15, c), L)] = v[15][c]

    lax.fori_loop(0, EV_PER_W // L, lambda m, car: (block(m), car)[1],
                  jnp.int32(0))

    copies = []
    for p in range(6):
        copies.append(pltpu.async_copy(
            out_v.at[pl.ds(p * CHUNK, CHUNK)],
            out_hbm.at[pl.ds(p * PLANE + wbase, CHUNK)], sem))
    for cp in copies:
        cp.wait()


@jax.jit
def _reorder(xf):
    mesh = plsc.VectorSubcoreMesh(core_axis_name="c", subcore_axis_name="s",
                                  num_cores=NC, num_subcores=NS)
    return pl.kernel(
        _sc_body,
        out_type=jax.ShapeDtypeStruct((TOTAL,), jnp.float32),
        mesh=mesh,
        scratch_types=[
            pltpu.VMEM((W_WORDS,), jnp.float32),
            pltpu.VMEM((W_WORDS,), jnp.float32),
            pltpu.SemaphoreType.DMA,
        ],
        compiler_params=pltpu.CompilerParams(needs_layout_passes=False),
    )(xf)


def kernel(inputs):
    xf = (inputs.transpose(2, 1, 0).reshape(3, 2, 8, 128, 128)
          .transpose(0, 1, 3, 2, 4).reshape(TOTAL))
    of = _reorder(xf)
    return (of.reshape(3, 2, 128, 8, 128).transpose(0, 1, 3, 2, 4)
            .reshape(3, 16, N_EVENTS).transpose(2, 1, 0))
